# Initial kernel scaffold; baseline (speedup 1.0000x reference)
import jax
import jax.numpy as jnp
from jax.experimental import pallas as pl


def kernel(ins, batch, dim, phi_W1, phi_b1, phi_W2, phi_b2, rho_W1, rho_b1, rho_W2, rho_b2):
    raise NotImplementedError("write your pallas kernel here")


# speedup vs baseline: 3.0505x; 3.0505x over previous
import functools

import jax
import jax.numpy as jnp
from jax.experimental import pallas as pl
from jax.experimental.pallas import tpu as pltpu

N_NODES = 10000
N_EDGES = 320000
D = 128
B = 1280
W = 128


def _scatter_kernel(ins_ref, ids_ref, w1_ref, b1_ref, acc_ref, cnt_ref):
    step = pl.program_id(0)

    @pl.when(step == 0)
    def _init():
        acc_ref[...] = jnp.zeros_like(acc_ref)
        cnt_ref[...] = jnp.zeros_like(cnt_ref)

    x = ins_ref[...]
    h = jnp.maximum(
        jnp.dot(x, w1_ref[...], preferred_element_type=jnp.float32)
        + b1_ref[...],
        0.0,
    )

    ids = ids_ref[0]
    lo = jnp.min(ids)
    hi = jnp.max(ids)

    col = jax.lax.broadcasted_iota(jnp.int32, (B, W), 1)
    ones_b = jnp.ones((B, 1), jnp.float32)

    def pass_body(pos):
        base = jnp.minimum((pos // 8) * 8, N_NODES - W)
        valid = (ids >= pos) & (ids < base + W)
        rel = ids - base
        oh = jnp.where(valid & (rel == col), 1.0, 0.0)
        seg = jax.lax.dot_general(
            oh, h, (((0,), (0,)), ((), ())),
            preferred_element_type=jnp.float32,
        )
        acc_ref[pl.ds(base, W), :] += seg
        cnt_ref[pl.ds(base, W), :] += jax.lax.dot_general(
            oh, ones_b, (((0,), (0,)), ((), ())),
            preferred_element_type=jnp.float32,
        )
        return base + W

    jax.lax.while_loop(lambda p: p <= hi, pass_body, lo)


def _finish_kernel(acc_ref, cnt_ref, w2_ref, b2_ref,
                   rw1_ref, rb1_ref, rw2_ref, rb2_ref, out_ref):
    c = cnt_ref[...]
    g = acc_ref[...] / jnp.maximum(c, 1.0)
    hm = (
        jnp.dot(g, w2_ref[...], preferred_element_type=jnp.float32)
        + b2_ref[...] * (c > 0)
    )
    h1 = jnp.maximum(
        jnp.dot(hm, rw1_ref[...], preferred_element_type=jnp.float32)
        + rb1_ref[...],
        0.0,
    )
    out_ref[...] = (
        jnp.dot(h1, rw2_ref[...], preferred_element_type=jnp.float32)
        + rb2_ref[...]
    )


@functools.partial(jax.jit, static_argnames=("interpret",))
def _run(ins, batch, phi_W1, phi_b1, phi_W2, phi_b2,
         rho_W1, rho_b1, rho_W2, rho_b2, interpret=False):
    ids3 = jnp.asarray(batch, jnp.int32).reshape(N_EDGES // B, B, 1)
    acc, cnt = pl.pallas_call(
        _scatter_kernel,
        grid=(N_EDGES // B,),
        in_specs=[
            pl.BlockSpec((B, D), lambda i: (i, 0)),
            pl.BlockSpec((1, B, 1), lambda i: (i, 0, 0)),
            pl.BlockSpec((D, D), lambda i: (0, 0)),
            pl.BlockSpec((1, D), lambda i: (0, 0)),
        ],
        out_specs=[
            pl.BlockSpec((N_NODES, D), lambda i: (0, 0)),
            pl.BlockSpec((N_NODES, 1), lambda i: (0, 0)),
        ],
        out_shape=[
            jax.ShapeDtypeStruct((N_NODES, D), jnp.float32),
            jax.ShapeDtypeStruct((N_NODES, 1), jnp.float32),
        ],
        compiler_params=pltpu.CompilerParams(
            dimension_semantics=("arbitrary",),
        ),
        interpret=interpret,
    )(ins, ids3, phi_W1, phi_b1.reshape(1, D))

    R = 1000
    out = pl.pallas_call(
        _finish_kernel,
        grid=(N_NODES // R,),
        in_specs=[
            pl.BlockSpec((R, D), lambda i: (i, 0)),
            pl.BlockSpec((R, 1), lambda i: (i, 0)),
            pl.BlockSpec((D, D), lambda i: (0, 0)),
            pl.BlockSpec((1, D), lambda i: (0, 0)),
            pl.BlockSpec((D, D), lambda i: (0, 0)),
            pl.BlockSpec((1, D), lambda i: (0, 0)),
            pl.BlockSpec((D, D), lambda i: (0, 0)),
            pl.BlockSpec((1, D), lambda i: (0, 0)),
        ],
        out_specs=pl.BlockSpec((R, D), lambda i: (i, 0)),
        out_shape=jax.ShapeDtypeStruct((N_NODES, D), jnp.float32),
        interpret=interpret,
    )(acc, cnt, phi_W2, phi_b2.reshape(1, D),
      rho_W1, rho_b1.reshape(1, D), rho_W2, rho_b2.reshape(1, D))
    return out


def kernel(ins, batch, dim, phi_W1, phi_b1, phi_W2, phi_b2,
           rho_W1, rho_b1, rho_W2, rho_b2):
    return _run(ins, batch, phi_W1, phi_b1, phi_W2, phi_b2,
                rho_W1, rho_b1, rho_W2, rho_b2)


# speedup vs baseline: 4.1415x; 1.3577x over previous
import functools

import jax
import jax.numpy as jnp
from jax.experimental import pallas as pl
from jax.experimental.pallas import tpu as pltpu

N_NODES = 10000
N_EDGES = 320000
D = 128
B = 2560
W = 128


def _scatter_kernel(bounds_ref, ins_ref, ids_ref, w1_ref, b1_ref, acc_ref):
    step = pl.program_id(0)

    @pl.when(step == 0)
    def _init():
        acc_ref[...] = jnp.zeros_like(acc_ref)

    x = ins_ref[...].astype(jnp.bfloat16)
    h = jnp.maximum(
        jnp.dot(x, w1_ref[...].astype(jnp.bfloat16),
                preferred_element_type=jnp.float32)
        + b1_ref[...],
        0.0,
    ).astype(jnp.bfloat16)
    lane = jax.lax.broadcasted_iota(jnp.int32, (B, W), 1)
    ones_col = (lane == 0).astype(jnp.bfloat16)
    hc = jnp.concatenate([h, ones_col], axis=1)

    ids = ids_ref[0]
    lo = bounds_ref[0, step]
    hi = bounds_ref[1, step]

    def pass_body(p):
        base = pl.multiple_of(p, 8)
        rel = ids - base
        oh = (rel == lane).astype(jnp.bfloat16)
        seg = jax.lax.dot_general(
            oh, hc, (((0,), (0,)), ((), ())),
            preferred_element_type=jnp.float32,
        )
        acc_ref[pl.ds(base, W), :] += seg
        return base + W

    jax.lax.while_loop(lambda p: p <= hi, pass_body, (lo // 8) * 8)


def _finish_kernel(acc_ref, w2_ref, b2_ref,
                   rw1_ref, rb1_ref, rw2_ref, rb2_ref, out_ref):
    a = acc_ref[...]
    c = a[:, D:D + 1]
    g = a[:, :D] / jnp.maximum(c, 1.0)
    hm = (
        jnp.dot(g, w2_ref[...], preferred_element_type=jnp.float32)
        + b2_ref[...] * (c > 0)
    )
    h1 = jnp.maximum(
        jnp.dot(hm, rw1_ref[...], preferred_element_type=jnp.float32)
        + rb1_ref[...],
        0.0,
    )
    out_ref[...] = (
        jnp.dot(h1, rw2_ref[...], preferred_element_type=jnp.float32)
        + rb2_ref[...]
    )


@functools.partial(jax.jit, static_argnames=("interpret",))
def _run(ins, batch, phi_W1, phi_b1, phi_W2, phi_b2,
         rho_W1, rho_b1, rho_W2, rho_b2, interpret=False):
    nb = N_EDGES // B
    ids = jnp.asarray(batch, jnp.int32)
    ids3 = ids.reshape(nb, B, 1)
    bounds = jnp.stack([ids[0::B], ids[B - 1::B]])
    n_pad = N_NODES + W
    acc = pl.pallas_call(
        _scatter_kernel,
        grid_spec=pltpu.PrefetchScalarGridSpec(
            num_scalar_prefetch=1,
            grid=(nb,),
            in_specs=[
                pl.BlockSpec((B, D), lambda i, s: (i, 0)),
                pl.BlockSpec((1, B, 1), lambda i, s: (i, 0, 0)),
                pl.BlockSpec((D, D), lambda i, s: (0, 0)),
                pl.BlockSpec((1, D), lambda i, s: (0, 0)),
            ],
            out_specs=pl.BlockSpec((n_pad, D + W), lambda i, s: (0, 0)),
        ),
        out_shape=jax.ShapeDtypeStruct((n_pad, D + W), jnp.float32),
        compiler_params=pltpu.CompilerParams(
            dimension_semantics=("arbitrary",),
        ),
        interpret=interpret,
    )(bounds, ins, ids3, phi_W1, phi_b1.reshape(1, D))

    R = 1000
    out = pl.pallas_call(
        _finish_kernel,
        grid=(N_NODES // R,),
        in_specs=[
            pl.BlockSpec((R, D + W), lambda i: (i, 0)),
            pl.BlockSpec((D, D), lambda i: (0, 0)),
            pl.BlockSpec((1, D), lambda i: (0, 0)),
            pl.BlockSpec((D, D), lambda i: (0, 0)),
            pl.BlockSpec((1, D), lambda i: (0, 0)),
            pl.BlockSpec((D, D), lambda i: (0, 0)),
            pl.BlockSpec((1, D), lambda i: (0, 0)),
        ],
        out_specs=pl.BlockSpec((R, D), lambda i: (i, 0)),
        out_shape=jax.ShapeDtypeStruct((N_NODES, D), jnp.float32),
        interpret=interpret,
    )(acc, phi_W2, phi_b2.reshape(1, D),
      rho_W1, rho_b1.reshape(1, D), rho_W2, rho_b2.reshape(1, D))
    return out


def kernel(ins, batch, dim, phi_W1, phi_b1, phi_W2, phi_b2,
           rho_W1, rho_b1, rho_W2, rho_b2):
    return _run(ins, batch, phi_W1, phi_b1, phi_W2, phi_b2,
                rho_W1, rho_b1, rho_W2, rho_b2)
